# Initial kernel scaffold; baseline (speedup 1.0000x reference)
import jax
import jax.numpy as jnp
from jax.experimental import pallas as pl


def kernel(x, edge_index, edge_type, batch, W1, root1, b1, W2, root2, b2):
    raise NotImplementedError("write your pallas kernel here")


# speedup vs baseline: 2.5256x; 2.5256x over previous
import functools

import jax
import jax.numpy as jnp
from jax import lax
from jax.experimental import pallas as pl
from jax.experimental.pallas import tpu as pltpu
from jax.experimental.pallas import tpu_sc as plsc

N = 50000
E = 800000
R = 4
HID = 128
NG = 64
ROWS = R * N

NTILES = 16
EPT = E // NTILES
STRIPE = ROWS // NTILES
CH = 80
NCH = EPT // CH
VB = 2000
NVB = N // VB


def _sc_agg_body(S, table, srcs, rows, zeros, out, idx_s, idx_r, buf, acc, sem):
    cid = lax.axis_index("c")
    sid = lax.axis_index("s")
    ebase = sid * EPT
    for p in range(S):
        t = cid * S + p
        pltpu.sync_copy(zeros, acc.at[pl.ds(sid * STRIPE, STRIPE)])
        plsc.subcore_barrier()

        def chunk(c, carry):
            off = pl.multiple_of(ebase + c * CH, 8)
            pltpu.sync_copy(srcs.at[t, pl.ds(off, CH)], idx_s)
            pltpu.sync_copy(rows.at[pl.ds(off, CH)], idx_r)
            pltpu.async_copy(table.at[idx_s], buf, sem).wait()
            pltpu.sync_copy(buf, acc.at[idx_r], add=True)
            return carry

        lax.fori_loop(0, NCH, chunk, 0)
        plsc.subcore_barrier()
        pltpu.sync_copy(acc.at[pl.ds(sid * STRIPE, STRIPE)],
                        out.at[pl.ds(sid * STRIPE, STRIPE), t])
        plsc.subcore_barrier()


def _make_sc_agg(S):
    mesh = plsc.VectorSubcoreMesh(core_axis_name="c", subcore_axis_name="s")
    return pl.kernel(
        functools.partial(_sc_agg_body, S),
        out_type=jax.ShapeDtypeStruct((ROWS, 2 * S, 8), jnp.float32),
        mesh=mesh,
        scratch_types=[
            pltpu.VMEM((CH,), jnp.int32),
            pltpu.VMEM((CH,), jnp.int32),
            pltpu.VMEM((CH, 8), jnp.float32),
            pltpu.VMEM_SHARED((ROWS, 8), jnp.float32),
            pltpu.SemaphoreType.DMA,
        ],
        compiler_params=pltpu.CompilerParams(use_tc_tiling_on_sc=False),
    )


def _tc1_body(x_ref, a1_ref, w1_ref, r1_ref, b1_ref, out_ref):
    acc = jnp.dot(x_ref[...], r1_ref[...],
                  preferred_element_type=jnp.float32) + b1_ref[...]
    for r in range(R):
        a = a1_ref[r]
        cnt = a[:, 15:16]
        m = jnp.dot(a, w1_ref[r], preferred_element_type=jnp.float32)
        acc = acc + m / jnp.maximum(cnt, 1.0)
    out_ref[...] = jnp.maximum(acc, 0.0)


def _tc2_body(nvb, h1_ref, a2_ref, a1_ref, w2_ref, r2_ref, b2_ref, batch_ref,
              out_s, out_c):
    i = pl.program_id(0)
    acc = jnp.dot(h1_ref[...], r2_ref[...],
                  preferred_element_type=jnp.float32) + b2_ref[...]
    for r in range(R):
        cnt = a1_ref[r][:, 15:16]
        m = jnp.dot(a2_ref[r], w2_ref[r], preferred_element_type=jnp.float32)
        acc = acc + m / jnp.maximum(cnt, 1.0)
    h2 = jnp.maximum(acc, 0.0)
    b = batch_ref[0]
    onehot = (lax.broadcasted_iota(jnp.int32, (NG, VB), 0) == b
              ).astype(jnp.float32)
    sums = jnp.dot(onehot, h2, preferred_element_type=jnp.float32)
    cnts = jnp.dot(onehot, jnp.ones((VB, 8), jnp.float32),
                   preferred_element_type=jnp.float32)

    @pl.when(i == 0)
    def _():
        out_s[...] = jnp.zeros_like(out_s)
        out_c[...] = jnp.zeros_like(out_c)

    out_s[...] += sums
    out_c[...] += cnts

    @pl.when(i == nvb - 1)
    def _():
        out_s[...] = out_s[...] / jnp.maximum(out_c[...][:, 0:1], 1.0)


def kernel(x, edge_index, edge_type, batch, W1, root1, b1, W2, root2, b2):
    f32 = jnp.float32
    src = edge_index[0]
    dst = edge_index[1]
    rows = edge_type * N + dst

    x_aug = jnp.concatenate([x, jnp.ones((N, 1), f32)], axis=1)
    t1 = x_aug.reshape(2 * N, 8)
    srcs1 = src[None, :] * 2 + jnp.arange(2, dtype=jnp.int32)[:, None]
    zeros = jnp.zeros((STRIPE, 8), f32)
    a1 = _make_sc_agg(1)(t1, srcs1, rows, zeros)
    a1v = a1.reshape(R, N, 16)

    w1_aug = jnp.concatenate([W1, jnp.zeros((R, 1, HID), f32)], axis=1)
    r1_aug = jnp.concatenate([root1, jnp.zeros((1, HID), f32)], axis=0)
    h1 = pl.pallas_call(
        _tc1_body,
        grid=(NVB,),
        in_specs=[
            pl.BlockSpec((VB, 16), lambda i: (i, 0)),
            pl.BlockSpec((R, VB, 16), lambda i: (0, i, 0)),
            pl.BlockSpec((R, 16, HID), lambda i: (0, 0, 0)),
            pl.BlockSpec((16, HID), lambda i: (0, 0)),
            pl.BlockSpec((1, HID), lambda i: (0, 0)),
        ],
        out_specs=pl.BlockSpec((VB, HID), lambda i: (i, 0)),
        out_shape=jax.ShapeDtypeStruct((N, HID), f32),
    )(x_aug, a1v, w1_aug, r1_aug, b1.reshape(1, HID))

    t2 = h1.reshape(16 * N, 8)
    srcs2 = src[None, :] * 16 + jnp.arange(16, dtype=jnp.int32)[:, None]
    a2 = _make_sc_agg(8)(t2, srcs2, rows, zeros)
    a2v = a2.reshape(R, N, HID)

    batch3 = batch.reshape(NVB, 1, VB)
    pooled, _ = pl.pallas_call(
        functools.partial(_tc2_body, NVB),
        grid=(NVB,),
        in_specs=[
            pl.BlockSpec((VB, HID), lambda i: (i, 0)),
            pl.BlockSpec((R, VB, HID), lambda i: (0, i, 0)),
            pl.BlockSpec((R, VB, 16), lambda i: (0, i, 0)),
            pl.BlockSpec((R, HID, HID), lambda i: (0, 0, 0)),
            pl.BlockSpec((HID, HID), lambda i: (0, 0)),
            pl.BlockSpec((1, HID), lambda i: (0, 0)),
            pl.BlockSpec((1, 1, VB), lambda i: (i, 0, 0)),
        ],
        out_specs=[
            pl.BlockSpec((NG, HID), lambda i: (0, 0)),
            pl.BlockSpec((NG, 8), lambda i: (0, 0)),
        ],
        out_shape=[
            jax.ShapeDtypeStruct((NG, HID), f32),
            jax.ShapeDtypeStruct((NG, 8), f32),
        ],
    )(h1, a2v, a1v, W2, root2, b2.reshape(1, HID), batch3)
    return pooled


# speedup vs baseline: 8.7557x; 3.4668x over previous
import functools

import jax
import jax.numpy as jnp
from jax import lax
from jax.experimental import pallas as pl
from jax.experimental.pallas import tpu as pltpu
from jax.experimental.pallas import tpu_sc as plsc

N = 50000
E = 800000
R = 4
HID = 128
NG = 64
ROWS = R * N

NTILES = 16
EPT = E // NTILES
STRIPE = ROWS // NTILES
CH = 2000
NCH = EPT // CH
VB = 2000
NVB = N // VB


def _sc_agg_body(S, table, srcs, rows, zeros, out, idx_s, idx_r, buf, acc, sem):
    cid = lax.axis_index("c")
    sid = lax.axis_index("s")
    ebase = sid * EPT
    for p in range(S):
        t = cid * S + p
        pltpu.sync_copy(zeros, acc.at[pl.ds(sid * STRIPE, STRIPE)])
        plsc.subcore_barrier()

        def chunk(c, carry):
            off = pl.multiple_of(ebase + c * CH, 8)
            pltpu.sync_copy(srcs.at[t, pl.ds(off, CH)], idx_s)
            pltpu.sync_copy(rows.at[pl.ds(off, CH)], idx_r)
            pltpu.async_copy(table.at[idx_s], buf, sem).wait()
            pltpu.sync_copy(buf, acc.at[idx_r], add=True)
            return carry

        lax.fori_loop(0, NCH, chunk, 0)
        plsc.subcore_barrier()
        pltpu.sync_copy(acc.at[pl.ds(sid * STRIPE, STRIPE)],
                        out.at[pl.ds(sid * STRIPE, STRIPE), t])
        plsc.subcore_barrier()


def _make_sc_agg(S):
    mesh = plsc.VectorSubcoreMesh(core_axis_name="c", subcore_axis_name="s")
    return pl.kernel(
        functools.partial(_sc_agg_body, S),
        out_type=jax.ShapeDtypeStruct((ROWS, 2 * S, 8), jnp.float32),
        mesh=mesh,
        scratch_types=[
            pltpu.VMEM((CH,), jnp.int32),
            pltpu.VMEM((CH,), jnp.int32),
            pltpu.VMEM((CH, 8), jnp.float32),
            pltpu.VMEM_SHARED((ROWS, 8), jnp.float32),
            pltpu.SemaphoreType.DMA,
        ],
        compiler_params=pltpu.CompilerParams(use_tc_tiling_on_sc=False),
    )


def _tc1_body(x_ref, a1_ref, w1_ref, r1_ref, b1_ref, out_ref):
    acc = jnp.dot(x_ref[...], r1_ref[...],
                  preferred_element_type=jnp.float32) + b1_ref[...]
    for r in range(R):
        a = a1_ref[r]
        cnt = a[:, 15:16]
        m = jnp.dot(a, w1_ref[r], preferred_element_type=jnp.float32)
        acc = acc + m / jnp.maximum(cnt, 1.0)
    out_ref[...] = jnp.maximum(acc, 0.0)


def _tc2_body(nvb, h1_ref, a2_ref, a1_ref, w2_ref, r2_ref, b2_ref, batch_ref,
              out_s, out_c):
    i = pl.program_id(0)
    acc = jnp.dot(h1_ref[...], r2_ref[...],
                  preferred_element_type=jnp.float32) + b2_ref[...]
    for r in range(R):
        cnt = a1_ref[r][:, 15:16]
        m = jnp.dot(a2_ref[r], w2_ref[r], preferred_element_type=jnp.float32)
        acc = acc + m / jnp.maximum(cnt, 1.0)
    h2 = jnp.maximum(acc, 0.0)
    b = batch_ref[0]
    onehot = (lax.broadcasted_iota(jnp.int32, (NG, VB), 0) == b
              ).astype(jnp.float32)
    sums = jnp.dot(onehot, h2, preferred_element_type=jnp.float32)
    cnts = jnp.dot(onehot, jnp.ones((VB, 8), jnp.float32),
                   preferred_element_type=jnp.float32)

    @pl.when(i == 0)
    def _():
        out_s[...] = jnp.zeros_like(out_s)
        out_c[...] = jnp.zeros_like(out_c)

    out_s[...] += sums
    out_c[...] += cnts

    @pl.when(i == nvb - 1)
    def _():
        out_s[...] = out_s[...] / jnp.maximum(out_c[...][:, 0:1], 1.0)


def kernel(x, edge_index, edge_type, batch, W1, root1, b1, W2, root2, b2):
    f32 = jnp.float32
    src = edge_index[0]
    dst = edge_index[1]
    rows = edge_type * N + dst

    x_aug = jnp.concatenate([x, jnp.ones((N, 1), f32)], axis=1)
    t1 = x_aug.reshape(2 * N, 8)
    srcs1 = src[None, :] * 2 + jnp.arange(2, dtype=jnp.int32)[:, None]
    zeros = jnp.zeros((STRIPE, 8), f32)
    a1 = _make_sc_agg(1)(t1, srcs1, rows, zeros)
    a1v = a1.reshape(R, N, 16)

    w1_aug = jnp.concatenate([W1, jnp.zeros((R, 1, HID), f32)], axis=1)
    r1_aug = jnp.concatenate([root1, jnp.zeros((1, HID), f32)], axis=0)
    h1 = pl.pallas_call(
        _tc1_body,
        grid=(NVB,),
        in_specs=[
            pl.BlockSpec((VB, 16), lambda i: (i, 0)),
            pl.BlockSpec((R, VB, 16), lambda i: (0, i, 0)),
            pl.BlockSpec((R, 16, HID), lambda i: (0, 0, 0)),
            pl.BlockSpec((16, HID), lambda i: (0, 0)),
            pl.BlockSpec((1, HID), lambda i: (0, 0)),
        ],
        out_specs=pl.BlockSpec((VB, HID), lambda i: (i, 0)),
        out_shape=jax.ShapeDtypeStruct((N, HID), f32),
    )(x_aug, a1v, w1_aug, r1_aug, b1.reshape(1, HID))

    t2 = h1.reshape(16 * N, 8)
    srcs2 = src[None, :] * 16 + jnp.arange(16, dtype=jnp.int32)[:, None]
    a2 = _make_sc_agg(8)(t2, srcs2, rows, zeros)
    a2v = a2.reshape(R, N, HID)

    batch3 = batch.reshape(NVB, 1, VB)
    pooled, _ = pl.pallas_call(
        functools.partial(_tc2_body, NVB),
        grid=(NVB,),
        in_specs=[
            pl.BlockSpec((VB, HID), lambda i: (i, 0)),
            pl.BlockSpec((R, VB, HID), lambda i: (0, i, 0)),
            pl.BlockSpec((R, VB, 16), lambda i: (0, i, 0)),
            pl.BlockSpec((R, HID, HID), lambda i: (0, 0, 0)),
            pl.BlockSpec((HID, HID), lambda i: (0, 0)),
            pl.BlockSpec((1, HID), lambda i: (0, 0)),
            pl.BlockSpec((1, 1, VB), lambda i: (i, 0, 0)),
        ],
        out_specs=[
            pl.BlockSpec((NG, HID), lambda i: (0, 0)),
            pl.BlockSpec((NG, 8), lambda i: (0, 0)),
        ],
        out_shape=[
            jax.ShapeDtypeStruct((NG, HID), f32),
            jax.ShapeDtypeStruct((NG, 8), f32),
        ],
    )(h1, a2v, a1v, W2, root2, b2.reshape(1, HID), batch3)
    return pooled


# speedup vs baseline: 12.8466x; 1.4672x over previous
import functools

import jax
import jax.numpy as jnp
from jax import lax
from jax.experimental import pallas as pl
from jax.experimental.pallas import tpu as pltpu
from jax.experimental.pallas import tpu_sc as plsc

N = 50000
E = 800000
R = 4
HID = 128
NG = 64
ROWS = R * N

NTILES = 16
EPT = E // NTILES
STRIPE = ROWS // NTILES
CH = 1000
NCH = EPT // CH
VB = 2000
NVB = N // VB


def _sc_agg_body(S, table, srcs, rows, zeros, out, idx_s, idx_r, buf, acc, sems):
    cid = lax.axis_index("c")
    sid = lax.axis_index("s")
    cbase = sid * NCH
    for p in range(S):
        t = cid * S + p
        pltpu.sync_copy(zeros, acc.at[pl.ds(sid * STRIPE, STRIPE)])
        plsc.subcore_barrier()

        pltpu.sync_copy(srcs.at[t, cbase], idx_s.at[0])
        pltpu.sync_copy(rows.at[cbase], idx_r.at[0])
        pltpu.async_copy(table.at[idx_s.at[0]], buf.at[0], sems.at[0])

        def chunk(c, carry):
            b = lax.rem(c, 2)
            nb = lax.rem(c + 1, 2)

            @pl.when(c + 1 < NCH)
            def _():
                pltpu.sync_copy(srcs.at[t, cbase + c + 1], idx_s.at[nb])
                pltpu.sync_copy(rows.at[cbase + c + 1], idx_r.at[nb])

            pltpu.make_async_copy(table.at[idx_s.at[b]], buf.at[b],
                                  sems.at[b]).wait()

            @pl.when(c + 1 < NCH)
            def _():
                pltpu.async_copy(table.at[idx_s.at[nb]], buf.at[nb],
                                 sems.at[nb])

            pltpu.sync_copy(buf.at[b], acc.at[idx_r.at[b]], add=True)
            return carry

        lax.fori_loop(0, NCH, chunk, 0)
        plsc.subcore_barrier()
        pltpu.sync_copy(acc.at[pl.ds(sid * STRIPE, STRIPE)],
                        out.at[pl.ds(sid * STRIPE, STRIPE), t])
        plsc.subcore_barrier()


def _make_sc_agg(S):
    mesh = plsc.VectorSubcoreMesh(core_axis_name="c", subcore_axis_name="s")
    return pl.kernel(
        functools.partial(_sc_agg_body, S),
        out_type=jax.ShapeDtypeStruct((ROWS, 2 * S, 8), jnp.float32),
        mesh=mesh,
        scratch_types=[
            pltpu.VMEM((2, CH), jnp.int32),
            pltpu.VMEM((2, CH), jnp.int32),
            pltpu.VMEM((2, CH, 8), jnp.float32),
            pltpu.VMEM_SHARED((ROWS, 8), jnp.float32),
            pltpu.SemaphoreType.DMA((2,)),
        ],
        compiler_params=pltpu.CompilerParams(use_tc_tiling_on_sc=False),
    )


def _tc1_body(x_ref, a1_ref, w1_ref, r1_ref, b1_ref, out_ref):
    acc = jnp.dot(x_ref[...], r1_ref[...],
                  preferred_element_type=jnp.float32) + b1_ref[...]
    for r in range(R):
        a = a1_ref[r]
        cnt = a[:, 15:16]
        m = jnp.dot(a, w1_ref[r], preferred_element_type=jnp.float32)
        acc = acc + m / jnp.maximum(cnt, 1.0)
    out_ref[...] = jnp.maximum(acc, 0.0)


def _tc2_body(nvb, h1_ref, a2_ref, a1_ref, w2_ref, r2_ref, b2_ref, batch_ref,
              out_s, out_c):
    i = pl.program_id(0)
    acc = jnp.dot(h1_ref[...], r2_ref[...],
                  preferred_element_type=jnp.float32) + b2_ref[...]
    for r in range(R):
        cnt = a1_ref[r][:, 15:16]
        m = jnp.dot(a2_ref[r], w2_ref[r], preferred_element_type=jnp.float32)
        acc = acc + m / jnp.maximum(cnt, 1.0)
    h2 = jnp.maximum(acc, 0.0)
    b = batch_ref[0]
    onehot = (lax.broadcasted_iota(jnp.int32, (NG, VB), 0) == b
              ).astype(jnp.float32)
    sums = jnp.dot(onehot, h2, preferred_element_type=jnp.float32)
    cnts = jnp.dot(onehot, jnp.ones((VB, 8), jnp.float32),
                   preferred_element_type=jnp.float32)

    @pl.when(i == 0)
    def _():
        out_s[...] = jnp.zeros_like(out_s)
        out_c[...] = jnp.zeros_like(out_c)

    out_s[...] += sums
    out_c[...] += cnts

    @pl.when(i == nvb - 1)
    def _():
        out_s[...] = out_s[...] / jnp.maximum(out_c[...][:, 0:1], 1.0)


def kernel(x, edge_index, edge_type, batch, W1, root1, b1, W2, root2, b2):
    f32 = jnp.float32
    src = edge_index[0]
    dst = edge_index[1]
    rows = edge_type * N + dst

    x_aug = jnp.concatenate([x, jnp.ones((N, 1), f32)], axis=1)
    t1 = x_aug.reshape(2 * N, 8)
    srcs1 = (src[None, :] * 2
             + jnp.arange(2, dtype=jnp.int32)[:, None]).reshape(2, E // CH, CH)
    rows = rows.reshape(E // CH, CH)
    zeros = jnp.zeros((STRIPE, 8), f32)
    a1 = _make_sc_agg(1)(t1, srcs1, rows, zeros)
    a1v = a1.reshape(R, N, 16)

    w1_aug = jnp.concatenate([W1, jnp.zeros((R, 1, HID), f32)], axis=1)
    r1_aug = jnp.concatenate([root1, jnp.zeros((1, HID), f32)], axis=0)
    h1 = pl.pallas_call(
        _tc1_body,
        grid=(NVB,),
        in_specs=[
            pl.BlockSpec((VB, 16), lambda i: (i, 0)),
            pl.BlockSpec((R, VB, 16), lambda i: (0, i, 0)),
            pl.BlockSpec((R, 16, HID), lambda i: (0, 0, 0)),
            pl.BlockSpec((16, HID), lambda i: (0, 0)),
            pl.BlockSpec((1, HID), lambda i: (0, 0)),
        ],
        out_specs=pl.BlockSpec((VB, HID), lambda i: (i, 0)),
        out_shape=jax.ShapeDtypeStruct((N, HID), f32),
    )(x_aug, a1v, w1_aug, r1_aug, b1.reshape(1, HID))

    t2 = h1.reshape(16 * N, 8)
    srcs2 = (src[None, :] * 16
             + jnp.arange(16, dtype=jnp.int32)[:, None]).reshape(16, E // CH, CH)
    a2 = _make_sc_agg(8)(t2, srcs2, rows, zeros)
    a2v = a2.reshape(R, N, HID)

    batch3 = batch.reshape(NVB, 1, VB)
    pooled, _ = pl.pallas_call(
        functools.partial(_tc2_body, NVB),
        grid=(NVB,),
        in_specs=[
            pl.BlockSpec((VB, HID), lambda i: (i, 0)),
            pl.BlockSpec((R, VB, HID), lambda i: (0, i, 0)),
            pl.BlockSpec((R, VB, 16), lambda i: (0, i, 0)),
            pl.BlockSpec((R, HID, HID), lambda i: (0, 0, 0)),
            pl.BlockSpec((HID, HID), lambda i: (0, 0)),
            pl.BlockSpec((1, HID), lambda i: (0, 0)),
            pl.BlockSpec((1, 1, VB), lambda i: (i, 0, 0)),
        ],
        out_specs=[
            pl.BlockSpec((NG, HID), lambda i: (0, 0)),
            pl.BlockSpec((NG, 8), lambda i: (0, 0)),
        ],
        out_shape=[
            jax.ShapeDtypeStruct((NG, HID), f32),
            jax.ShapeDtypeStruct((NG, 8), f32),
        ],
    )(h1, a2v, a1v, W2, root2, b2.reshape(1, HID), batch3)
    return pooled


# speedup vs baseline: 15.4342x; 1.2014x over previous
import functools

import jax
import jax.numpy as jnp
from jax import lax
from jax.experimental import pallas as pl
from jax.experimental.pallas import tpu as pltpu
from jax.experimental.pallas import tpu_sc as plsc

N = 50000
E = 800000
R = 4
HID = 128
NG = 64
ROWS = R * N

NTILES = 16
EPT = E // NTILES
STRIPE = ROWS // NTILES
CH = 400
NBUF = 6
NIDX = 8
NCH = EPT // CH
VB = 2000
NVB = N // VB


def _sc_agg_body(S, table, srcs, rows, zeros, out, idx_s, idx_r, buf, acc,
                 gsem, ssem, isem):
    cid = lax.axis_index("c")
    sid = lax.axis_index("s")
    cbase = sid * NCH
    for p in range(S):
        t = cid * S + p
        pltpu.sync_copy(zeros, acc.at[pl.ds(sid * STRIPE, STRIPE)])
        plsc.subcore_barrier()

        def idx_copy(k):
            ks = lax.rem(k, NIDX)
            return (pltpu.make_async_copy(srcs.at[t, cbase + k],
                                          idx_s.at[ks], isem.at[ks]),
                    pltpu.make_async_copy(rows.at[cbase + k],
                                          idx_r.at[ks], isem.at[ks]))

        def gat_copy(k):
            kb = lax.rem(k, NBUF)
            return pltpu.make_async_copy(
                table.at[idx_s.at[lax.rem(k, NIDX)]], buf.at[kb], gsem.at[kb])

        def sct_copy(k):
            kb = lax.rem(k, NBUF)
            return pltpu.make_async_copy(
                buf.at[kb], acc.at[idx_r.at[lax.rem(k, NIDX)]], ssem.at[kb])

        for k in range(min(5, NCH)):
            a, bcopy = idx_copy(k)
            a.start(), bcopy.start()
        for k in range(min(3, NCH)):
            a, bcopy = idx_copy(k)
            a.wait(), bcopy.wait()
            gat_copy(k).start()

        def chunk(c, carry):
            gat_copy(c).wait()
            sct_copy(c).start(add=True)

            @pl.when(c >= 3)
            def _():
                sct_copy(c - 3).wait()

            @pl.when(c + 3 < NCH)
            def _():
                a, bcopy = idx_copy(c + 3)
                a.wait(), bcopy.wait()
                gat_copy(c + 3).start()

            @pl.when(c + 5 < NCH)
            def _():
                a, bcopy = idx_copy(c + 5)
                a.start(), bcopy.start()

            return carry

        lax.fori_loop(0, NCH, chunk, 0)
        for k in range(max(NCH - 3, 0), NCH):
            sct_copy(k).wait()
        plsc.subcore_barrier()
        pltpu.sync_copy(acc.at[pl.ds(sid * STRIPE, STRIPE)],
                        out.at[pl.ds(sid * STRIPE, STRIPE), t])
        plsc.subcore_barrier()


def _make_sc_agg(S):
    mesh = plsc.VectorSubcoreMesh(core_axis_name="c", subcore_axis_name="s")
    return pl.kernel(
        functools.partial(_sc_agg_body, S),
        out_type=jax.ShapeDtypeStruct((ROWS, 2 * S, 8), jnp.float32),
        mesh=mesh,
        scratch_types=[
            pltpu.VMEM((NIDX, CH), jnp.int32),
            pltpu.VMEM((NIDX, CH), jnp.int32),
            pltpu.VMEM((NBUF, CH, 8), jnp.float32),
            pltpu.VMEM_SHARED((ROWS, 8), jnp.float32),
            pltpu.SemaphoreType.DMA((NBUF,)),
            pltpu.SemaphoreType.DMA((NBUF,)),
            pltpu.SemaphoreType.DMA((NIDX,)),
        ],
        compiler_params=pltpu.CompilerParams(use_tc_tiling_on_sc=False),
    )


def _tc1_body(x_ref, a1_ref, w1_ref, r1_ref, b1_ref, out_ref):
    acc = jnp.dot(x_ref[...], r1_ref[...],
                  preferred_element_type=jnp.float32) + b1_ref[...]
    for r in range(R):
        a = a1_ref[r]
        cnt = a[:, 15:16]
        m = jnp.dot(a, w1_ref[r], preferred_element_type=jnp.float32)
        acc = acc + m / jnp.maximum(cnt, 1.0)
    out_ref[...] = jnp.maximum(acc, 0.0)


def _tc2_body(nvb, h1_ref, a2_ref, a1_ref, w2_ref, r2_ref, b2_ref, batch_ref,
              out_s, out_c):
    i = pl.program_id(0)
    acc = jnp.dot(h1_ref[...], r2_ref[...],
                  preferred_element_type=jnp.float32) + b2_ref[...]
    for r in range(R):
        cnt = a1_ref[r][:, 15:16]
        m = jnp.dot(a2_ref[r], w2_ref[r], preferred_element_type=jnp.float32)
        acc = acc + m / jnp.maximum(cnt, 1.0)
    h2 = jnp.maximum(acc, 0.0)
    b = batch_ref[0]
    onehot = (lax.broadcasted_iota(jnp.int32, (NG, VB), 0) == b
              ).astype(jnp.float32)
    sums = jnp.dot(onehot, h2, preferred_element_type=jnp.float32)
    cnts = jnp.dot(onehot, jnp.ones((VB, 8), jnp.float32),
                   preferred_element_type=jnp.float32)

    @pl.when(i == 0)
    def _():
        out_s[...] = jnp.zeros_like(out_s)
        out_c[...] = jnp.zeros_like(out_c)

    out_s[...] += sums
    out_c[...] += cnts

    @pl.when(i == nvb - 1)
    def _():
        out_s[...] = out_s[...] / jnp.maximum(out_c[...][:, 0:1], 1.0)


def kernel(x, edge_index, edge_type, batch, W1, root1, b1, W2, root2, b2):
    f32 = jnp.float32
    src = edge_index[0]
    dst = edge_index[1]
    rows = edge_type * N + dst

    x_aug = jnp.concatenate([x, jnp.ones((N, 1), f32)], axis=1)
    t1 = x_aug.reshape(2 * N, 8)
    srcs1 = (src[None, :] * 2
             + jnp.arange(2, dtype=jnp.int32)[:, None]).reshape(2, E // CH, CH)
    rows = rows.reshape(E // CH, CH)
    zeros = jnp.zeros((STRIPE, 8), f32)
    a1 = _make_sc_agg(1)(t1, srcs1, rows, zeros)
    a1v = a1.reshape(R, N, 16)

    w1_aug = jnp.concatenate([W1, jnp.zeros((R, 1, HID), f32)], axis=1)
    r1_aug = jnp.concatenate([root1, jnp.zeros((1, HID), f32)], axis=0)
    h1 = pl.pallas_call(
        _tc1_body,
        grid=(NVB,),
        in_specs=[
            pl.BlockSpec((VB, 16), lambda i: (i, 0)),
            pl.BlockSpec((R, VB, 16), lambda i: (0, i, 0)),
            pl.BlockSpec((R, 16, HID), lambda i: (0, 0, 0)),
            pl.BlockSpec((16, HID), lambda i: (0, 0)),
            pl.BlockSpec((1, HID), lambda i: (0, 0)),
        ],
        out_specs=pl.BlockSpec((VB, HID), lambda i: (i, 0)),
        out_shape=jax.ShapeDtypeStruct((N, HID), f32),
    )(x_aug, a1v, w1_aug, r1_aug, b1.reshape(1, HID))

    t2 = h1.reshape(16 * N, 8)
    srcs2 = (src[None, :] * 16
             + jnp.arange(16, dtype=jnp.int32)[:, None]).reshape(16, E // CH, CH)
    a2 = _make_sc_agg(8)(t2, srcs2, rows, zeros)
    a2v = a2.reshape(R, N, HID)

    batch3 = batch.reshape(NVB, 1, VB)
    pooled, _ = pl.pallas_call(
        functools.partial(_tc2_body, NVB),
        grid=(NVB,),
        in_specs=[
            pl.BlockSpec((VB, HID), lambda i: (i, 0)),
            pl.BlockSpec((R, VB, HID), lambda i: (0, i, 0)),
            pl.BlockSpec((R, VB, 16), lambda i: (0, i, 0)),
            pl.BlockSpec((R, HID, HID), lambda i: (0, 0, 0)),
            pl.BlockSpec((HID, HID), lambda i: (0, 0)),
            pl.BlockSpec((1, HID), lambda i: (0, 0)),
            pl.BlockSpec((1, 1, VB), lambda i: (i, 0, 0)),
        ],
        out_specs=[
            pl.BlockSpec((NG, HID), lambda i: (0, 0)),
            pl.BlockSpec((NG, 8), lambda i: (0, 0)),
        ],
        out_shape=[
            jax.ShapeDtypeStruct((NG, HID), f32),
            jax.ShapeDtypeStruct((NG, 8), f32),
        ],
    )(h1, a2v, a1v, W2, root2, b2.reshape(1, HID), batch3)
    return pooled


# speedup vs baseline: 15.5401x; 1.0069x over previous
import functools

import jax
import jax.numpy as jnp
from jax import lax
from jax.experimental import pallas as pl
from jax.experimental.pallas import tpu as pltpu
from jax.experimental.pallas import tpu_sc as plsc

N = 50000
E = 800000
R = 4
HID = 128
NG = 64
ROWS = R * N

NTILES = 16
EPT = E // NTILES
STRIPE = ROWS // NTILES
CH = 400
NBUF = 6
NIDX = 8
NCH = EPT // CH
VB = 2000
NVB = N // VB


def _sc_agg_body(S, table, srcs, rows, zeros, out, idx_s, idx_r, buf, acc,
                 gsem, isem):
    cid = lax.axis_index("c")
    sid = lax.axis_index("s")
    cbase = sid * NCH
    for p in range(S):
        t = cid * S + p
        pltpu.sync_copy(zeros, acc.at[pl.ds(sid * STRIPE, STRIPE)])
        plsc.subcore_barrier()

        def idx_copy(k):
            ks = lax.rem(k, NIDX)
            return (pltpu.make_async_copy(srcs.at[t, cbase + k],
                                          idx_s.at[ks], isem.at[ks]),
                    pltpu.make_async_copy(rows.at[cbase + k],
                                          idx_r.at[ks], isem.at[ks]))

        def gat_copy(k):
            kb = lax.rem(k, NBUF)
            return pltpu.make_async_copy(
                table.at[idx_s.at[lax.rem(k, NIDX)]], buf.at[kb], gsem.at[kb])

        for k in range(min(5, NCH)):
            a, bcopy = idx_copy(k)
            a.start(), bcopy.start()
        for k in range(min(3, NCH)):
            a, bcopy = idx_copy(k)
            a.wait(), bcopy.wait()
            gat_copy(k).start()

        def chunk(c, carry):
            gat_copy(c).wait()

            @pl.when(c + 3 < NCH)
            def _():
                a, bcopy = idx_copy(c + 3)
                a.wait(), bcopy.wait()
                gat_copy(c + 3).start()

            @pl.when(c + 5 < NCH)
            def _():
                a, bcopy = idx_copy(c + 5)
                a.start(), bcopy.start()

            pltpu.sync_copy(buf.at[lax.rem(c, NBUF)],
                            acc.at[idx_r.at[lax.rem(c, NIDX)]], add=True)
            return carry

        lax.fori_loop(0, NCH, chunk, 0)
        plsc.subcore_barrier()
        pltpu.sync_copy(acc.at[pl.ds(sid * STRIPE, STRIPE)],
                        out.at[pl.ds(sid * STRIPE, STRIPE), t])
        plsc.subcore_barrier()


def _make_sc_agg(S):
    mesh = plsc.VectorSubcoreMesh(core_axis_name="c", subcore_axis_name="s")
    return pl.kernel(
        functools.partial(_sc_agg_body, S),
        out_type=jax.ShapeDtypeStruct((ROWS, 2 * S, 8), jnp.float32),
        mesh=mesh,
        scratch_types=[
            pltpu.VMEM((NIDX, CH), jnp.int32),
            pltpu.VMEM((NIDX, CH), jnp.int32),
            pltpu.VMEM((NBUF, CH, 8), jnp.float32),
            pltpu.VMEM_SHARED((ROWS, 8), jnp.float32),
            pltpu.SemaphoreType.DMA((NBUF,)),
            pltpu.SemaphoreType.DMA((NIDX,)),
        ],
        compiler_params=pltpu.CompilerParams(use_tc_tiling_on_sc=False),
    )


def _tc1_body(x_ref, a1_ref, w1_ref, r1_ref, b1_ref, out_ref):
    acc = jnp.dot(x_ref[...], r1_ref[...],
                  preferred_element_type=jnp.float32) + b1_ref[...]
    for r in range(R):
        a = a1_ref[r]
        cnt = a[:, 15:16]
        m = jnp.dot(a, w1_ref[r], preferred_element_type=jnp.float32)
        acc = acc + m / jnp.maximum(cnt, 1.0)
    out_ref[...] = jnp.maximum(acc, 0.0)


def _tc2_body(nvb, h1_ref, a2_ref, a1_ref, w2_ref, r2_ref, b2_ref, batch_ref,
              out_s, out_c):
    i = pl.program_id(0)
    acc = jnp.dot(h1_ref[...], r2_ref[...],
                  preferred_element_type=jnp.float32) + b2_ref[...]
    for r in range(R):
        cnt = a1_ref[r][:, 15:16]
        m = jnp.dot(a2_ref[r], w2_ref[r], preferred_element_type=jnp.float32)
        acc = acc + m / jnp.maximum(cnt, 1.0)
    h2 = jnp.maximum(acc, 0.0)
    b = batch_ref[0]
    onehot = (lax.broadcasted_iota(jnp.int32, (NG, VB), 0) == b
              ).astype(jnp.float32)
    sums = jnp.dot(onehot, h2, preferred_element_type=jnp.float32)
    cnts = jnp.dot(onehot, jnp.ones((VB, 8), jnp.float32),
                   preferred_element_type=jnp.float32)

    @pl.when(i == 0)
    def _():
        out_s[...] = jnp.zeros_like(out_s)
        out_c[...] = jnp.zeros_like(out_c)

    out_s[...] += sums
    out_c[...] += cnts

    @pl.when(i == nvb - 1)
    def _():
        out_s[...] = out_s[...] / jnp.maximum(out_c[...][:, 0:1], 1.0)


def kernel(x, edge_index, edge_type, batch, W1, root1, b1, W2, root2, b2):
    f32 = jnp.float32
    src = edge_index[0]
    dst = edge_index[1]
    rows = edge_type * N + dst

    x_aug = jnp.concatenate([x, jnp.ones((N, 1), f32)], axis=1)
    t1 = x_aug.reshape(2 * N, 8)
    srcs1 = (src[None, :] * 2
             + jnp.arange(2, dtype=jnp.int32)[:, None]).reshape(2, E // CH, CH)
    rows = rows.reshape(E // CH, CH)
    zeros = jnp.zeros((STRIPE, 8), f32)
    a1 = _make_sc_agg(1)(t1, srcs1, rows, zeros)
    a1v = a1.reshape(R, N, 16)

    w1_aug = jnp.concatenate([W1, jnp.zeros((R, 1, HID), f32)], axis=1)
    r1_aug = jnp.concatenate([root1, jnp.zeros((1, HID), f32)], axis=0)
    h1 = pl.pallas_call(
        _tc1_body,
        grid=(NVB,),
        in_specs=[
            pl.BlockSpec((VB, 16), lambda i: (i, 0)),
            pl.BlockSpec((R, VB, 16), lambda i: (0, i, 0)),
            pl.BlockSpec((R, 16, HID), lambda i: (0, 0, 0)),
            pl.BlockSpec((16, HID), lambda i: (0, 0)),
            pl.BlockSpec((1, HID), lambda i: (0, 0)),
        ],
        out_specs=pl.BlockSpec((VB, HID), lambda i: (i, 0)),
        out_shape=jax.ShapeDtypeStruct((N, HID), f32),
    )(x_aug, a1v, w1_aug, r1_aug, b1.reshape(1, HID))

    t2 = h1.reshape(16 * N, 8)
    srcs2 = (src[None, :] * 16
             + jnp.arange(16, dtype=jnp.int32)[:, None]).reshape(16, E // CH, CH)
    a2 = _make_sc_agg(8)(t2, srcs2, rows, zeros)
    a2v = a2.reshape(R, N, HID)

    batch3 = batch.reshape(NVB, 1, VB)
    pooled, _ = pl.pallas_call(
        functools.partial(_tc2_body, NVB),
        grid=(NVB,),
        in_specs=[
            pl.BlockSpec((VB, HID), lambda i: (i, 0)),
            pl.BlockSpec((R, VB, HID), lambda i: (0, i, 0)),
            pl.BlockSpec((R, VB, 16), lambda i: (0, i, 0)),
            pl.BlockSpec((R, HID, HID), lambda i: (0, 0, 0)),
            pl.BlockSpec((HID, HID), lambda i: (0, 0)),
            pl.BlockSpec((1, HID), lambda i: (0, 0)),
            pl.BlockSpec((1, 1, VB), lambda i: (i, 0, 0)),
        ],
        out_specs=[
            pl.BlockSpec((NG, HID), lambda i: (0, 0)),
            pl.BlockSpec((NG, 8), lambda i: (0, 0)),
        ],
        out_shape=[
            jax.ShapeDtypeStruct((NG, HID), f32),
            jax.ShapeDtypeStruct((NG, 8), f32),
        ],
    )(h1, a2v, a1v, W2, root2, b2.reshape(1, HID), batch3)
    return pooled
